# baseline (speedup 1.0000x reference)
import functools

import jax
import jax.numpy as jnp
from jax import lax
from jax.experimental import pallas as pl
from jax.experimental.pallas import tpu as pltpu
from jax.experimental.pallas import tpu_sc as plsc

NUM_USERS = 1000000
NUM_MOVIES = 100000
NUM_CATS = 1000
U_DIM = 64
M_DIM = 32
C_DIM = 32
B = 16384
L = 20
MARGIN = 0.1

def _tc_transpose_users(eut):
    cols = 1664

    def body(in_ref, out_ref):
        xt = in_ref[...].T
        out_ref[...] = jnp.concatenate([xt, xt], axis=1)

    return pl.pallas_call(
        body,
        grid=((NUM_USERS + cols - 1) // cols,),
        in_specs=[pl.BlockSpec((U_DIM, cols), lambda i: (0, i))],
        out_specs=pl.BlockSpec((cols, 128), lambda i: (i, 0)),
        out_shape=jax.ShapeDtypeStruct((NUM_USERS, 128), jnp.float32),
    )(eut)


_INFO = plsc.get_sparse_core_info()
NC = _INFO.num_cores
NS = _INFO.num_subcores
LANES = _INFO.num_lanes
NW = NC * NS
RPW = B // NW
NG = RPW // LANES
HD = U_DIM // 2


@functools.partial(
    pl.kernel,
    out_type=jax.ShapeDtypeStruct((B,), jnp.float32),
    mesh=plsc.VectorSubcoreMesh(core_axis_name="c", subcore_axis_name="s"),
    compiler_params=pltpu.CompilerParams(
        needs_layout_passes=False, use_tc_tiling_on_sc=False),
    scratch_types=[
        pltpu.VMEM((RPW,), jnp.int32),
        pltpu.VMEM((RPW,), jnp.int32),
        pltpu.VMEM((L, RPW), jnp.int32),
        pltpu.VMEM((HD * RPW,), jnp.int32),
        pltpu.VMEM((8 * RPW,), jnp.int32),
        pltpu.VMEM((HD * RPW,), jnp.float32),
        pltpu.VMEM((8 * RPW, 8), jnp.float32),
        pltpu.VMEM((C_DIM, NUM_CATS), jnp.float32),
        pltpu.VMEM((RPW,), jnp.float32),
        pltpu.VMEM((RPW,), jnp.float32),
        pltpu.VMEM((RPW,), jnp.float32),
        pltpu.SemaphoreType.DMA,
    ],
)
def _sc_forward(uid_hbm, mid_hbm, cidx_hbm, eu_hbm, em_hbm, ec_hbm,
                bu_hbm, bm_hbm, out_hbm,
                uid_v, mid_v, cidx_v, idx_v, uidx_v, m_elem, u_oct, ctab_v,
                bu_v, bm_v, out_v, sem):
    wid = lax.axis_index("s") * NC + lax.axis_index("c")
    base = wid * RPW
    iota = lax.iota(jnp.int32, LANES)

    pltpu.sync_copy(uid_hbm.at[pl.ds(base, RPW)], uid_v)
    pltpu.sync_copy(mid_hbm.at[pl.ds(base, RPW)], mid_v)
    pltpu.sync_copy(cidx_hbm.at[:, pl.ds(base, RPW)], cidx_v)
    pltpu.sync_copy(ec_hbm, ctab_v)
    pltpu.async_copy(bu_hbm.at[uid_v], bu_v, sem).wait()
    pltpu.async_copy(bm_hbm.at[mid_v], bm_v, sem).wait()

    def gm_body(g, carry):
        g16 = g * LANES
        ids = mid_v[pl.ds(g16, LANES)]
        for d in range(HD):
            idx_v[pl.ds(d * RPW + g16, LANES)] = (
                ids + jnp.int32(d * NUM_MOVIES))
        return carry

    lax.fori_loop(0, NG, gm_body, 0)
    pltpu.async_copy(em_hbm.at[idx_v], m_elem, sem).wait()

    def gu_body(g, carry):
        g16 = g * LANES
        ids8 = uid_v[pl.ds(g16, LANES)] * 16
        for d8 in range(8):
            uidx_v[pl.ds(d8 * RPW + g16, LANES)] = ids8 + jnp.int32(d8)
        return carry

    lax.fori_loop(0, NG, gu_body, 0)
    pltpu.async_copy(eu_hbm.at[uidx_v], u_oct, sem).wait()

    def pa_body(g, carry):
        g16 = g * LANES
        p0 = jnp.zeros((LANES,), jnp.float32)
        p1 = jnp.zeros((LANES,), jnp.float32)
        for d in range(0, HD, 2):
            rows = jnp.full((LANES,), (d >> 3) * RPW + g16, jnp.int32) + iota
            u0 = plsc.load_gather(
                u_oct, [rows, jnp.full((LANES,), d & 7, jnp.int32)])
            u1 = plsc.load_gather(
                u_oct, [rows, jnp.full((LANES,), (d + 1) & 7, jnp.int32)])
            o = d * RPW + g16
            p0 = p0 + u0 * m_elem[pl.ds(o, LANES)]
            p1 = p1 + u1 * m_elem[pl.ds(o + RPW, LANES)]
        out_v[pl.ds(g16, LANES)] = p0 + p1
        return carry

    lax.fori_loop(0, NG, pa_body, 0)

    def pb_body(g, carry):
        g16 = g * LANES
        ids = [cidx_v[sl, pl.ds(g16, LANES)] for sl in range(L)]
        cnt = jnp.zeros((LANES,), jnp.float32)
        for sl in range(L):
            cnt = cnt + jnp.where(ids[sl] != 0, 1.0, 0.0)

        def d_body(d, accs):
            a0, a1, a2, a3 = accs
            dvec = jnp.full((LANES,), d, jnp.int32)
            rows = (jnp.full((LANES,), g16, jnp.int32) + iota
                    + ((d >> 3) + 4) * RPW)
            par = jnp.full((LANES,), d & 7, jnp.int32)
            u2 = plsc.load_gather(u_oct, [rows, par])
            for sl in range(0, L, 4):
                a0 = a0 + plsc.load_gather(ctab_v, [dvec, ids[sl]]) * u2
                a1 = a1 + plsc.load_gather(ctab_v, [dvec, ids[sl + 1]]) * u2
                a2 = a2 + plsc.load_gather(ctab_v, [dvec, ids[sl + 2]]) * u2
                a3 = a3 + plsc.load_gather(ctab_v, [dvec, ids[sl + 3]]) * u2
            return (a0, a1, a2, a3)

        z = jnp.zeros((LANES,), jnp.float32)
        a0, a1, a2, a3 = lax.fori_loop(0, HD, d_body, (z, z, z, z))
        pc = (a0 + a1) + (a2 + a3)
        x = (out_v[pl.ds(g16, LANES)] + pc / jnp.maximum(cnt, 1.0)
             + bu_v[pl.ds(g16, LANES)] + bm_v[pl.ds(g16, LANES)])
        sig = 1.0 / (1.0 + jnp.exp(-x))
        out_v[pl.ds(g16, LANES)] = sig * (1.0 + 2 * MARGIN) - MARGIN
        return carry

    lax.fori_loop(0, NG, pb_body, 0)
    pltpu.sync_copy(out_v, out_hbm.at[pl.ds(base, RPW)])


def kernel(user_id, movie_id, movie_categories, emb_users, emb_movies,
           emb_movie_cats, bias_user, bias_movie):
    uid = user_id.astype(jnp.int32)
    mid = movie_id.astype(jnp.int32)
    cidx = movie_categories.astype(jnp.int32).T
    eu = _tc_transpose_users(emb_users.T).reshape(16000000, 8)
    em = emb_movies.T.reshape(-1)
    ec = emb_movie_cats.T
    bu = bias_user.T.reshape(-1)
    bm = bias_movie.T.reshape(-1)
    return _sc_forward(uid, mid, cidx, eu, em, ec, bu, bm)


# speedup vs baseline: 1.6381x; 1.6381x over previous
import functools

import jax
import jax.numpy as jnp
from jax import lax
from jax.experimental import pallas as pl
from jax.experimental.pallas import tpu as pltpu
from jax.experimental.pallas import tpu_sc as plsc

NUM_USERS = 1000000
NUM_MOVIES = 100000
NUM_CATS = 1000
U_DIM = 64
M_DIM = 32
C_DIM = 32
B = 16384
L = 20
MARGIN = 0.1

def _tc_transpose_users(eut):
    cols = 13312

    def body(in_ref, out_ref):
        xt = in_ref[...].T
        out_ref[...] = jnp.concatenate([xt, xt], axis=1)

    return pl.pallas_call(
        body,
        grid=((NUM_USERS + cols - 1) // cols,),
        in_specs=[pl.BlockSpec((U_DIM, cols), lambda i: (0, i))],
        out_specs=pl.BlockSpec((cols, 128), lambda i: (i, 0)),
        out_shape=jax.ShapeDtypeStruct((NUM_USERS, 128), jnp.float32),
    )(eut)


_INFO = plsc.get_sparse_core_info()
NC = _INFO.num_cores
NS = _INFO.num_subcores
LANES = _INFO.num_lanes
NW = NC * NS
RPW = B // NW
NG = RPW // LANES
HD = U_DIM // 2


@functools.partial(
    pl.kernel,
    out_type=jax.ShapeDtypeStruct((B,), jnp.float32),
    mesh=plsc.VectorSubcoreMesh(core_axis_name="c", subcore_axis_name="s"),
    compiler_params=pltpu.CompilerParams(
        needs_layout_passes=False, use_tc_tiling_on_sc=False),
    scratch_types=[
        pltpu.VMEM((RPW,), jnp.int32),
        pltpu.VMEM((RPW,), jnp.int32),
        pltpu.VMEM((L, RPW), jnp.int32),
        pltpu.VMEM((HD * RPW,), jnp.int32),
        pltpu.VMEM((8 * RPW,), jnp.int32),
        pltpu.VMEM((HD * RPW,), jnp.float32),
        pltpu.VMEM((8 * RPW, 8), jnp.float32),
        pltpu.VMEM((C_DIM, NUM_CATS), jnp.float32),
        pltpu.VMEM((RPW,), jnp.float32),
        pltpu.VMEM((RPW,), jnp.float32),
        pltpu.VMEM((RPW,), jnp.float32),
        pltpu.SemaphoreType.DMA,
    ],
)
def _sc_forward(uid_hbm, mid_hbm, cidx_hbm, eu_hbm, em_hbm, ec_hbm,
                bu_hbm, bm_hbm, out_hbm,
                uid_v, mid_v, cidx_v, idx_v, uidx_v, m_elem, u_oct, ctab_v,
                bu_v, bm_v, out_v, sem):
    wid = lax.axis_index("s") * NC + lax.axis_index("c")
    base = wid * RPW
    iota = lax.iota(jnp.int32, LANES)

    pltpu.sync_copy(uid_hbm.at[pl.ds(base, RPW)], uid_v)
    pltpu.sync_copy(mid_hbm.at[pl.ds(base, RPW)], mid_v)
    pltpu.sync_copy(cidx_hbm.at[:, pl.ds(base, RPW)], cidx_v)
    pltpu.sync_copy(ec_hbm, ctab_v)
    pltpu.async_copy(bu_hbm.at[uid_v], bu_v, sem).wait()
    pltpu.async_copy(bm_hbm.at[mid_v], bm_v, sem).wait()

    def gm_body(g, carry):
        g16 = g * LANES
        ids = mid_v[pl.ds(g16, LANES)]
        for d in range(HD):
            idx_v[pl.ds(d * RPW + g16, LANES)] = (
                ids + jnp.int32(d * NUM_MOVIES))
        return carry

    lax.fori_loop(0, NG, gm_body, 0)
    pltpu.async_copy(em_hbm.at[idx_v], m_elem, sem).wait()

    def gu_body(g, carry):
        g16 = g * LANES
        ids8 = uid_v[pl.ds(g16, LANES)] * 16
        for d8 in range(8):
            uidx_v[pl.ds(d8 * RPW + g16, LANES)] = ids8 + jnp.int32(d8)
        return carry

    lax.fori_loop(0, NG, gu_body, 0)
    pltpu.async_copy(eu_hbm.at[uidx_v], u_oct, sem).wait()

    def pa_body(g, carry):
        g16 = g * LANES
        p0 = jnp.zeros((LANES,), jnp.float32)
        p1 = jnp.zeros((LANES,), jnp.float32)
        for d in range(0, HD, 2):
            rows = jnp.full((LANES,), (d >> 3) * RPW + g16, jnp.int32) + iota
            u0 = plsc.load_gather(
                u_oct, [rows, jnp.full((LANES,), d & 7, jnp.int32)])
            u1 = plsc.load_gather(
                u_oct, [rows, jnp.full((LANES,), (d + 1) & 7, jnp.int32)])
            o = d * RPW + g16
            p0 = p0 + u0 * m_elem[pl.ds(o, LANES)]
            p1 = p1 + u1 * m_elem[pl.ds(o + RPW, LANES)]
        out_v[pl.ds(g16, LANES)] = p0 + p1
        return carry

    lax.fori_loop(0, NG, pa_body, 0)

    def pb_body(g, carry):
        g16 = g * LANES
        ids = [cidx_v[sl, pl.ds(g16, LANES)] for sl in range(L)]
        cnt = jnp.zeros((LANES,), jnp.float32)
        for sl in range(L):
            cnt = cnt + jnp.where(ids[sl] != 0, 1.0, 0.0)

        def d_body(d, accs):
            a0, a1, a2, a3 = accs
            dvec = jnp.full((LANES,), d, jnp.int32)
            rows = (jnp.full((LANES,), g16, jnp.int32) + iota
                    + ((d >> 3) + 4) * RPW)
            par = jnp.full((LANES,), d & 7, jnp.int32)
            u2 = plsc.load_gather(u_oct, [rows, par])
            for sl in range(0, L, 4):
                a0 = a0 + plsc.load_gather(ctab_v, [dvec, ids[sl]]) * u2
                a1 = a1 + plsc.load_gather(ctab_v, [dvec, ids[sl + 1]]) * u2
                a2 = a2 + plsc.load_gather(ctab_v, [dvec, ids[sl + 2]]) * u2
                a3 = a3 + plsc.load_gather(ctab_v, [dvec, ids[sl + 3]]) * u2
            return (a0, a1, a2, a3)

        z = jnp.zeros((LANES,), jnp.float32)
        a0, a1, a2, a3 = lax.fori_loop(0, HD, d_body, (z, z, z, z))
        pc = (a0 + a1) + (a2 + a3)
        x = (out_v[pl.ds(g16, LANES)] + pc / jnp.maximum(cnt, 1.0)
             + bu_v[pl.ds(g16, LANES)] + bm_v[pl.ds(g16, LANES)])
        sig = 1.0 / (1.0 + jnp.exp(-x))
        out_v[pl.ds(g16, LANES)] = sig * (1.0 + 2 * MARGIN) - MARGIN
        return carry

    lax.fori_loop(0, NG, pb_body, 0)
    pltpu.sync_copy(out_v, out_hbm.at[pl.ds(base, RPW)])


def kernel(user_id, movie_id, movie_categories, emb_users, emb_movies,
           emb_movie_cats, bias_user, bias_movie):
    uid = user_id.astype(jnp.int32)
    mid = movie_id.astype(jnp.int32)
    cidx = movie_categories.astype(jnp.int32).T
    eu = _tc_transpose_users(emb_users.T).reshape(16000000, 8)
    em = emb_movies.T.reshape(-1)
    ec = emb_movie_cats.T
    bu = bias_user.T.reshape(-1)
    bm = bias_movie.T.reshape(-1)
    return _sc_forward(uid, mid, cidx, eu, em, ec, bu, bm)


# speedup vs baseline: 1.7154x; 1.0472x over previous
import functools

import jax
import jax.numpy as jnp
from jax import lax
from jax.experimental import pallas as pl
from jax.experimental.pallas import tpu as pltpu
from jax.experimental.pallas import tpu_sc as plsc

NUM_USERS = 1000000
NUM_MOVIES = 100000
NUM_CATS = 1000
U_DIM = 64
M_DIM = 32
C_DIM = 32
B = 16384
L = 20
MARGIN = 0.1

def _tc_transpose_users(eut):
    cols = 26624
    chunk = 1664

    def body(in_ref, out_ref):
        for k in range(cols // chunk):
            xt = in_ref[:, pl.ds(k * chunk, chunk)].T
            out_ref[pl.ds(k * chunk, chunk), :] = (
                jnp.concatenate([xt, xt], axis=1))

    return pl.pallas_call(
        body,
        grid=((NUM_USERS + cols - 1) // cols,),
        in_specs=[pl.BlockSpec((U_DIM, cols), lambda i: (0, i))],
        out_specs=pl.BlockSpec((cols, 128), lambda i: (i, 0)),
        out_shape=jax.ShapeDtypeStruct((NUM_USERS, 128), jnp.float32),
    )(eut)


_INFO = plsc.get_sparse_core_info()
NC = _INFO.num_cores
NS = _INFO.num_subcores
LANES = _INFO.num_lanes
NW = NC * NS
RPW = B // NW
NG = RPW // LANES
HD = U_DIM // 2


@functools.partial(
    pl.kernel,
    out_type=jax.ShapeDtypeStruct((B,), jnp.float32),
    mesh=plsc.VectorSubcoreMesh(core_axis_name="c", subcore_axis_name="s"),
    compiler_params=pltpu.CompilerParams(
        needs_layout_passes=False, use_tc_tiling_on_sc=False),
    scratch_types=[
        pltpu.VMEM((RPW,), jnp.int32),
        pltpu.VMEM((RPW,), jnp.int32),
        pltpu.VMEM((L, RPW), jnp.int32),
        pltpu.VMEM((HD * RPW,), jnp.int32),
        pltpu.VMEM((8 * RPW,), jnp.int32),
        pltpu.VMEM((HD * RPW,), jnp.float32),
        pltpu.VMEM((8 * RPW, 8), jnp.float32),
        pltpu.VMEM((C_DIM, NUM_CATS), jnp.float32),
        pltpu.VMEM((RPW,), jnp.float32),
        pltpu.VMEM((RPW,), jnp.float32),
        pltpu.VMEM((RPW,), jnp.float32),
        pltpu.SemaphoreType.DMA,
    ],
)
def _sc_forward(uid_hbm, mid_hbm, cidx_hbm, eu_hbm, em_hbm, ec_hbm,
                bu_hbm, bm_hbm, out_hbm,
                uid_v, mid_v, cidx_v, idx_v, uidx_v, m_elem, u_oct, ctab_v,
                bu_v, bm_v, out_v, sem):
    wid = lax.axis_index("s") * NC + lax.axis_index("c")
    base = wid * RPW
    iota = lax.iota(jnp.int32, LANES)

    pltpu.sync_copy(uid_hbm.at[pl.ds(base, RPW)], uid_v)
    pltpu.sync_copy(mid_hbm.at[pl.ds(base, RPW)], mid_v)
    pltpu.sync_copy(cidx_hbm.at[:, pl.ds(base, RPW)], cidx_v)
    pltpu.sync_copy(ec_hbm, ctab_v)
    pltpu.async_copy(bu_hbm.at[uid_v], bu_v, sem).wait()
    pltpu.async_copy(bm_hbm.at[mid_v], bm_v, sem).wait()

    def gm_body(g, carry):
        g16 = g * LANES
        ids = mid_v[pl.ds(g16, LANES)]
        for d in range(HD):
            idx_v[pl.ds(d * RPW + g16, LANES)] = (
                ids + jnp.int32(d * NUM_MOVIES))
        return carry

    lax.fori_loop(0, NG, gm_body, 0)
    pltpu.async_copy(em_hbm.at[idx_v], m_elem, sem).wait()

    def gu_body(g, carry):
        g16 = g * LANES
        ids8 = uid_v[pl.ds(g16, LANES)] * 16
        for d8 in range(8):
            uidx_v[pl.ds(d8 * RPW + g16, LANES)] = ids8 + jnp.int32(d8)
        return carry

    lax.fori_loop(0, NG, gu_body, 0)
    pltpu.async_copy(eu_hbm.at[uidx_v], u_oct, sem).wait()

    def pa_body(g, carry):
        g16 = g * LANES
        p0 = jnp.zeros((LANES,), jnp.float32)
        p1 = jnp.zeros((LANES,), jnp.float32)
        for d in range(0, HD, 2):
            rows = jnp.full((LANES,), (d >> 3) * RPW + g16, jnp.int32) + iota
            u0 = plsc.load_gather(
                u_oct, [rows, jnp.full((LANES,), d & 7, jnp.int32)])
            u1 = plsc.load_gather(
                u_oct, [rows, jnp.full((LANES,), (d + 1) & 7, jnp.int32)])
            o = d * RPW + g16
            p0 = p0 + u0 * m_elem[pl.ds(o, LANES)]
            p1 = p1 + u1 * m_elem[pl.ds(o + RPW, LANES)]
        out_v[pl.ds(g16, LANES)] = p0 + p1
        return carry

    lax.fori_loop(0, NG, pa_body, 0)

    def pb_body(g, carry):
        g16 = g * LANES
        ids = [cidx_v[sl, pl.ds(g16, LANES)] for sl in range(L)]
        cnt = jnp.zeros((LANES,), jnp.float32)
        for sl in range(L):
            cnt = cnt + jnp.where(ids[sl] != 0, 1.0, 0.0)

        def d_body(d, accs):
            a0, a1, a2, a3 = accs
            dvec = jnp.full((LANES,), d, jnp.int32)
            rows = (jnp.full((LANES,), g16, jnp.int32) + iota
                    + ((d >> 3) + 4) * RPW)
            par = jnp.full((LANES,), d & 7, jnp.int32)
            u2 = plsc.load_gather(u_oct, [rows, par])
            for sl in range(0, L, 4):
                a0 = a0 + plsc.load_gather(ctab_v, [dvec, ids[sl]]) * u2
                a1 = a1 + plsc.load_gather(ctab_v, [dvec, ids[sl + 1]]) * u2
                a2 = a2 + plsc.load_gather(ctab_v, [dvec, ids[sl + 2]]) * u2
                a3 = a3 + plsc.load_gather(ctab_v, [dvec, ids[sl + 3]]) * u2
            return (a0, a1, a2, a3)

        z = jnp.zeros((LANES,), jnp.float32)
        a0, a1, a2, a3 = lax.fori_loop(0, HD, d_body, (z, z, z, z))
        pc = (a0 + a1) + (a2 + a3)
        x = (out_v[pl.ds(g16, LANES)] + pc / jnp.maximum(cnt, 1.0)
             + bu_v[pl.ds(g16, LANES)] + bm_v[pl.ds(g16, LANES)])
        sig = 1.0 / (1.0 + jnp.exp(-x))
        out_v[pl.ds(g16, LANES)] = sig * (1.0 + 2 * MARGIN) - MARGIN
        return carry

    lax.fori_loop(0, NG, pb_body, 0)
    pltpu.sync_copy(out_v, out_hbm.at[pl.ds(base, RPW)])


def kernel(user_id, movie_id, movie_categories, emb_users, emb_movies,
           emb_movie_cats, bias_user, bias_movie):
    uid = user_id.astype(jnp.int32)
    mid = movie_id.astype(jnp.int32)
    cidx = movie_categories.astype(jnp.int32).T
    eu = _tc_transpose_users(emb_users.T).reshape(16000000, 8)
    em = emb_movies.T.reshape(-1)
    ec = emb_movie_cats.T
    bu = bias_user.T.reshape(-1)
    bm = bias_movie.T.reshape(-1)
    return _sc_forward(uid, mid, cidx, eu, em, ec, bu, bm)
